# baseline (speedup 1.0000x reference)
import functools

import jax
import jax.numpy as jnp
from jax import lax
from jax.experimental import pallas as pl
from jax.experimental.pallas import tpu as pltpu
from jax.experimental.pallas import tpu_sc as plsc

B, L, D = 4096, 200, 64
KEY_V = 100000
N = B * L
NW = 32
TW = N // NW
C = 256
NG = C // 16
NCH = TW // C
NP = NCH // 2
EPS = 1e-5


def _sc_body(tid_hbm, nt_hbm, pk_hbm, cat_hbm,
             depth_hbm, sibling_hbm, ntype_hbm, gam_hbm, bet_hbm,
             out_hbm,
             depnt_tab, sib_tab, dep_tmp, nt_tmp, gam_v, bet_v,
             tidA, ntA, tidB, ntB, cidxA, cidxB, rowsA, rowsB, out_dma,
             pkA, pkB,
             sem_idx, sem_gA, sem_gB, sem_o):
    wid = lax.axis_index("s") * 2 + lax.axis_index("c")
    lanes = lax.iota(jnp.int32, 16)

    def csplat(v):
        return jnp.full((16,), v, jnp.int32)

    pltpu.sync_copy(depth_hbm, dep_tmp)
    pltpu.sync_copy(ntype_hbm, nt_tmp)
    pltpu.sync_copy(sibling_hbm, sib_tab)
    pltpu.sync_copy(gam_hbm, gam_v)
    pltpu.sync_copy(bet_hbm, bet_v)
    ntrows = [[plsc.load_gather(nt_tmp, [csplat(t), lanes + c * 16])
               for c in range(4)] for t in range(4)]

    def build_depnt(dep, _):
        for c in range(4):
            dchunk = plsc.load_gather(dep_tmp, [csplat(dep), lanes + c * 16])
            for t in range(4):
                plsc.store_scatter(depnt_tab,
                                   [csplat(dep * 4 + t), lanes + c * 16],
                                   dchunk + ntrows[t][c])
        return 0

    lax.fori_loop(0, 64, build_depnt, 0, unroll=False)

    def fire_idx(k, tid_v, ntv, pk):
        base = wid * TW + k * C
        pltpu.async_copy(tid_hbm.at[pl.ds(base, C)], tid_v, sem_idx)
        pltpu.async_copy(nt_hbm.at[pl.ds(base, C)], ntv, sem_idx)
        pltpu.async_copy(pk_hbm.at[pl.ds(base, C)], pk, sem_idx)

    def wait_idx(tid_v, ntv, pk):
        for r in (tid_v, ntv, pk):
            pltpu.make_async_copy(tid_hbm.at[pl.ds(0, C)], r, sem_idx).wait()

    def prep(tid_v, ntv, cidx):
        def prep_g(g, _):
            t = tid_v[pl.ds(g * 16, 16)]
            n = ntv[pl.ds(g * 16, 16)]
            is_key = (n == 0) | (n == 2)
            cidx[pl.ds(g * 16, 16)] = jnp.where(is_key, t, t + KEY_V)
            return 0

        lax.fori_loop(0, NG, prep_g, 0, unroll=True)

    GSPLIT = 4
    GQ = C // GSPLIT

    def fire_gather(cidx, rows, sem):
        for qq in range(GSPLIT):
            pltpu.async_copy(cat_hbm.at[cidx.at[pl.ds(qq * GQ, GQ)]],
                             rows.at[pl.ds(qq * GQ, GQ)], sem)

    def wait_gather(cidx, rows, sem):
        for qq in range(GSPLIT):
            pltpu.make_async_copy(cat_hbm.at[cidx.at[pl.ds(qq * GQ, GQ)]],
                                  rows.at[pl.ds(qq * GQ, GQ)], sem).wait()

    def wait_out():
        pltpu.make_async_copy(out_dma, out_hbm.at[pl.ds(0, C * D)], sem_o).wait()

    shuf_dnums = lax.GatherDimensionNumbers(
        offset_dims=(), collapsed_slice_dims=(0,), start_index_map=(0,))
    perms = {sh: jnp.reshape(lanes ^ sh, (16, 1)) for sh in (8, 4, 2, 1)}

    def lane_sum(v):
        for sh in (8, 4, 2, 1):
            v = v + lax.gather(v, perms[sh], shuf_dnums, (1,),
                               mode=lax.GatherScatterMode.PROMISE_IN_BOUNDS)
        return v

    def compute(k, rows, pk):
        g4 = [gam_v[pl.ds(c * 16, 16)] for c in range(4)]
        b4 = [bet_v[pl.ds(c * 16, 16)] for c in range(4)]

        def grp(g, _):
            w16 = pk[pl.ds(g * 16, 16)]
            dn16 = ((w16 >> 10) & 63) * 4 + (w16 & 3)
            sb16 = (w16 >> 2) & 255
            for u in range(16):
                tsp = csplat(g * 16 + u)
                osp = csplat((g * 16 + u) * D)
                dnsp = csplat(dn16[u])
                sbsp = csplat(sb16[u])
                xs = []
                for c in range(4):
                    col = lanes + c * 16
                    v = plsc.load_gather(rows, [tsp, col])
                    v = v + plsc.load_gather(depnt_tab, [dnsp, col])
                    v = v + plsc.load_gather(sib_tab, [sbsp, col])
                    xs.append(v)
                s = (xs[0] + xs[1]) + (xs[2] + xs[3])
                q = ((xs[0] * xs[0] + xs[1] * xs[1])
                     + (xs[2] * xs[2] + xs[3] * xs[3]))
                mu = lane_sum(s) * (1.0 / D)
                var = lane_sum(q) * (1.0 / D) - mu * mu + EPS
                y = lax.bitcast_convert_type(
                    jnp.int32(0x5F3759DF)
                    - lax.shift_right_arithmetic(
                        lax.bitcast_convert_type(var, jnp.int32), 1),
                    jnp.float32)
                y = y * (1.5 - 0.5 * var * y * y)
                y = y * (1.5 - 0.5 * var * y * y)
                y = y * (1.5 - 0.5 * var * y * y)
                for c in range(4):
                    rg = y * g4[c]
                    bc = b4[c] - mu * rg
                    plsc.store_scatter(out_dma, [osp + (lanes + c * 16)],
                                       xs[c] * rg + bc)
            return 0

        lax.fori_loop(0, NG, grp, 0, unroll=False)

        @pl.when(k > 0)
        def _():
            wait_out()

        base = (wid * TW + k * C) * D
        pltpu.async_copy(out_dma, out_hbm.at[pl.ds(base, C * D)], sem_o)

    fire_idx(0, tidA, ntA, pkA)
    wait_idx(tidA, ntA, pkA)
    prep(tidA, ntA, cidxA)
    fire_gather(cidxA, rowsA, sem_gA)
    fire_idx(1, tidB, ntB, pkB)

    def pair(kk, _):
        k0 = 2 * kk
        wait_idx(tidB, ntB, pkB)
        prep(tidB, ntB, cidxB)
        fire_gather(cidxB, rowsB, sem_gB)
        wait_gather(cidxA, rowsA, sem_gA)
        compute(k0, rowsA, pkA)

        @pl.when(kk < NP - 1)
        def _():
            fire_idx(k0 + 2, tidA, ntA, pkA)
            wait_idx(tidA, ntA, pkA)
            prep(tidA, ntA, cidxA)
            fire_gather(cidxA, rowsA, sem_gA)

        wait_gather(cidxB, rowsB, sem_gB)
        compute(k0 + 1, rowsB, pkB)

        @pl.when(kk < NP - 1)
        def _():
            fire_idx(k0 + 3, tidB, ntB, pkB)

        return 0

    lax.fori_loop(0, NP, pair, 0, unroll=False)
    wait_out()


_sc_embed = functools.partial(
    pl.kernel,
    out_type=jax.ShapeDtypeStruct((N * D,), jnp.float32),
    mesh=plsc.VectorSubcoreMesh(core_axis_name="c", subcore_axis_name="s"),
    compiler_params=pltpu.CompilerParams(
        needs_layout_passes=False, use_tc_tiling_on_sc=False),
    scratch_types=[
        pltpu.VMEM((256, D), jnp.float32),
        pltpu.VMEM((256, D), jnp.float32),
        pltpu.VMEM((64, D), jnp.float32),
        pltpu.VMEM((4, D), jnp.float32),
        pltpu.VMEM((D,), jnp.float32),
        pltpu.VMEM((D,), jnp.float32),
        pltpu.VMEM((C,), jnp.int32),
        pltpu.VMEM((C,), jnp.int32),
        pltpu.VMEM((C,), jnp.int32),
        pltpu.VMEM((C,), jnp.int32),
        pltpu.VMEM((C,), jnp.int32),
        pltpu.VMEM((C,), jnp.int32),
        pltpu.VMEM((C, D), jnp.float32),
        pltpu.VMEM((C, D), jnp.float32),
        pltpu.VMEM((C * D,), jnp.float32),
        pltpu.VMEM((C,), jnp.int32),
        pltpu.VMEM((C,), jnp.int32),
        pltpu.SemaphoreType.DMA,
        pltpu.SemaphoreType.DMA,
        pltpu.SemaphoreType.DMA,
        pltpu.SemaphoreType.DMA,
    ],
)(_sc_body)


def kernel(token_ids, node_types, depths, sibling_indices, key_table,
           value_table, depth_table, sibling_table, node_type_table,
           gamma, beta):
    tid = token_ids.reshape(N).astype(jnp.int32)
    nt = node_types.reshape(N).astype(jnp.int32)
    dep = depths.reshape(N).astype(jnp.int32)
    sib = sibling_indices.reshape(N).astype(jnp.int32)
    packed = ((tid & 1) << 16) | (dep << 10) | (sib << 2) | nt
    cat = jnp.concatenate([key_table.astype(jnp.float32),
                           value_table.astype(jnp.float32)], axis=0)
    out = _sc_embed(tid, nt, packed, cat,
                    depth_table.astype(jnp.float32),
                    sibling_table.astype(jnp.float32),
                    node_type_table.astype(jnp.float32),
                    gamma.astype(jnp.float32),
                    beta.astype(jnp.float32))
    return out.reshape(B, L, D)


# speedup vs baseline: 1.0174x; 1.0174x over previous
import functools

import jax
import jax.numpy as jnp
from jax import lax
from jax.experimental import pallas as pl
from jax.experimental.pallas import tpu as pltpu
from jax.experimental.pallas import tpu_sc as plsc

B, L, D = 4096, 200, 64
KEY_V = 100000
N = B * L
NW = 32
TW = N // NW
C = 256
NG = C // 16
NCH = TW // C
NP = NCH // 2
EPS = 1e-5


def _sc_body(tid_hbm, nt_hbm, pk_hbm, cat_hbm,
             depth_hbm, sibling_hbm, ntype_hbm, gam_hbm, bet_hbm,
             out_hbm,
             depnt_tab, sib_tab, dep_tmp, nt_tmp, gam_v, bet_v,
             tidA, ntA, tidB, ntB, cidxA, cidxB, rowsA, rowsB, out_dma,
             pkA, pkB,
             sem_idx, sem_gA, sem_gB, sem_o):
    wid = lax.axis_index("s") * 2 + lax.axis_index("c")
    lanes = lax.iota(jnp.int32, 16)

    def csplat(v):
        return jnp.full((16,), v, jnp.int32)

    pltpu.sync_copy(depth_hbm, dep_tmp)
    pltpu.sync_copy(ntype_hbm, nt_tmp)
    pltpu.sync_copy(sibling_hbm, sib_tab)
    pltpu.sync_copy(gam_hbm, gam_v)
    pltpu.sync_copy(bet_hbm, bet_v)
    ntrows = [[plsc.load_gather(nt_tmp, [csplat(t), lanes + c * 16])
               for c in range(4)] for t in range(4)]

    def build_depnt(dep, _):
        for c in range(4):
            dchunk = plsc.load_gather(dep_tmp, [csplat(dep), lanes + c * 16])
            for t in range(4):
                plsc.store_scatter(depnt_tab,
                                   [csplat(dep * 4 + t), lanes + c * 16],
                                   dchunk + ntrows[t][c])
        return 0

    lax.fori_loop(0, 64, build_depnt, 0, unroll=False)

    def fire_idx(k, tid_v, ntv, pk):
        base = wid * TW + k * C
        pltpu.async_copy(tid_hbm.at[pl.ds(base, C)], tid_v, sem_idx)
        pltpu.async_copy(nt_hbm.at[pl.ds(base, C)], ntv, sem_idx)
        pltpu.async_copy(pk_hbm.at[pl.ds(base, C)], pk, sem_idx)

    def wait_idx(tid_v, ntv, pk):
        for r in (tid_v, ntv, pk):
            pltpu.make_async_copy(tid_hbm.at[pl.ds(0, C)], r, sem_idx).wait()

    def prep(tid_v, ntv, cidx):
        def prep_g(g, _):
            t = tid_v[pl.ds(g * 16, 16)]
            n = ntv[pl.ds(g * 16, 16)]
            is_key = (n == 0) | (n == 2)
            cidx[pl.ds(g * 16, 16)] = jnp.where(is_key, t, t + KEY_V)
            return 0

        lax.fori_loop(0, NG, prep_g, 0, unroll=True)

    GSPLIT = 4
    GQ = C // GSPLIT

    def fire_gather(cidx, rows, sem):
        for qq in range(GSPLIT):
            pltpu.async_copy(cat_hbm.at[cidx.at[pl.ds(qq * GQ, GQ)]],
                             rows.at[pl.ds(qq * GQ, GQ)], sem)

    def wait_gather(cidx, rows, sem):
        for qq in range(GSPLIT):
            pltpu.make_async_copy(cat_hbm.at[cidx.at[pl.ds(qq * GQ, GQ)]],
                                  rows.at[pl.ds(qq * GQ, GQ)], sem).wait()

    def wait_out():
        pltpu.make_async_copy(out_dma, out_hbm.at[pl.ds(0, C * D)], sem_o).wait()

    def compute(k, rows, pk):
        g4 = [gam_v[pl.ds(c * 16, 16)] for c in range(4)]
        b4 = [bet_v[pl.ds(c * 16, 16)] for c in range(4)]

        def grp(g, _):
            w16 = pk[pl.ds(g * 16, 16)]
            dn16 = ((w16 >> 10) & 63) * 4 + (w16 & 3)
            sb16 = (w16 >> 2) & 255
            for u in range(16):
                tsp = csplat(g * 16 + u)
                osp = csplat((g * 16 + u) * D)
                dnsp = csplat(dn16[u])
                sbsp = csplat(sb16[u])
                xs = []
                for c in range(4):
                    col = lanes + c * 16
                    v = plsc.load_gather(rows, [tsp, col])
                    v = v + plsc.load_gather(depnt_tab, [dnsp, col])
                    v = v + plsc.load_gather(sib_tab, [sbsp, col])
                    xs.append(v)
                s = (xs[0] + xs[1]) + (xs[2] + xs[3])
                q = ((xs[0] * xs[0] + xs[1] * xs[1])
                     + (xs[2] * xs[2] + xs[3] * xs[3]))
                mu = jnp.full((16,), jnp.sum(s), jnp.float32) * (1.0 / D)
                msq = jnp.full((16,), jnp.sum(q), jnp.float32) * (1.0 / D)
                var = msq - mu * mu + EPS
                y = lax.bitcast_convert_type(
                    jnp.int32(0x5F3759DF)
                    - lax.shift_right_arithmetic(
                        lax.bitcast_convert_type(var, jnp.int32), 1),
                    jnp.float32)
                y = y * (1.5 - 0.5 * var * y * y)
                y = y * (1.5 - 0.5 * var * y * y)
                y = y * (1.5 - 0.5 * var * y * y)
                for c in range(4):
                    rg = y * g4[c]
                    bc = b4[c] - mu * rg
                    plsc.store_scatter(out_dma, [osp + (lanes + c * 16)],
                                       xs[c] * rg + bc)
            return 0

        lax.fori_loop(0, NG, grp, 0, unroll=False)

        @pl.when(k > 0)
        def _():
            wait_out()

        base = (wid * TW + k * C) * D
        pltpu.async_copy(out_dma, out_hbm.at[pl.ds(base, C * D)], sem_o)

    fire_idx(0, tidA, ntA, pkA)
    wait_idx(tidA, ntA, pkA)
    prep(tidA, ntA, cidxA)
    fire_gather(cidxA, rowsA, sem_gA)
    fire_idx(1, tidB, ntB, pkB)

    def pair(kk, _):
        k0 = 2 * kk
        wait_idx(tidB, ntB, pkB)
        prep(tidB, ntB, cidxB)
        fire_gather(cidxB, rowsB, sem_gB)
        wait_gather(cidxA, rowsA, sem_gA)
        compute(k0, rowsA, pkA)

        @pl.when(kk < NP - 1)
        def _():
            fire_idx(k0 + 2, tidA, ntA, pkA)
            wait_idx(tidA, ntA, pkA)
            prep(tidA, ntA, cidxA)
            fire_gather(cidxA, rowsA, sem_gA)

        wait_gather(cidxB, rowsB, sem_gB)
        compute(k0 + 1, rowsB, pkB)

        @pl.when(kk < NP - 1)
        def _():
            fire_idx(k0 + 3, tidB, ntB, pkB)

        return 0

    lax.fori_loop(0, NP, pair, 0, unroll=False)
    wait_out()


_sc_embed = functools.partial(
    pl.kernel,
    out_type=jax.ShapeDtypeStruct((N * D,), jnp.float32),
    mesh=plsc.VectorSubcoreMesh(core_axis_name="c", subcore_axis_name="s"),
    compiler_params=pltpu.CompilerParams(
        needs_layout_passes=False, use_tc_tiling_on_sc=False),
    scratch_types=[
        pltpu.VMEM((256, D), jnp.float32),
        pltpu.VMEM((256, D), jnp.float32),
        pltpu.VMEM((64, D), jnp.float32),
        pltpu.VMEM((4, D), jnp.float32),
        pltpu.VMEM((D,), jnp.float32),
        pltpu.VMEM((D,), jnp.float32),
        pltpu.VMEM((C,), jnp.int32),
        pltpu.VMEM((C,), jnp.int32),
        pltpu.VMEM((C,), jnp.int32),
        pltpu.VMEM((C,), jnp.int32),
        pltpu.VMEM((C,), jnp.int32),
        pltpu.VMEM((C,), jnp.int32),
        pltpu.VMEM((C, D), jnp.float32),
        pltpu.VMEM((C, D), jnp.float32),
        pltpu.VMEM((C * D,), jnp.float32),
        pltpu.VMEM((C,), jnp.int32),
        pltpu.VMEM((C,), jnp.int32),
        pltpu.SemaphoreType.DMA,
        pltpu.SemaphoreType.DMA,
        pltpu.SemaphoreType.DMA,
        pltpu.SemaphoreType.DMA,
    ],
)(_sc_body)


def kernel(token_ids, node_types, depths, sibling_indices, key_table,
           value_table, depth_table, sibling_table, node_type_table,
           gamma, beta):
    tid = token_ids.reshape(N).astype(jnp.int32)
    nt = node_types.reshape(N).astype(jnp.int32)
    dep = depths.reshape(N).astype(jnp.int32)
    sib = sibling_indices.reshape(N).astype(jnp.int32)
    packed = ((tid & 1) << 16) | (dep << 10) | (sib << 2) | nt
    cat = jnp.concatenate([key_table.astype(jnp.float32),
                           value_table.astype(jnp.float32)], axis=0)
    out = _sc_embed(tid, nt, packed, cat,
                    depth_table.astype(jnp.float32),
                    sibling_table.astype(jnp.float32),
                    node_type_table.astype(jnp.float32),
                    gamma.astype(jnp.float32),
                    beta.astype(jnp.float32))
    return out.reshape(B, L, D)


# speedup vs baseline: 1.0240x; 1.0065x over previous
import functools

import jax
import jax.numpy as jnp
from jax import lax
from jax.experimental import pallas as pl
from jax.experimental.pallas import tpu as pltpu
from jax.experimental.pallas import tpu_sc as plsc

B, L, D = 4096, 200, 64
KEY_V = 100000
N = B * L
NW = 32
TW = N // NW
C = 256
NG = C // 16
NCH = TW // C
NP = NCH // 2
EPS = 1e-5


def _sc_body(tid_hbm, nt_hbm, pk_hbm, cat_hbm,
             depth_hbm, sibling_hbm, ntype_hbm, gam_hbm, bet_hbm,
             out_hbm,
             depnt_tab, sib_tab, dep_tmp, nt_tmp, gam_v, bet_v,
             tidA, ntA, tidB, ntB, cidxA, cidxB, rowsA, rowsB, out_dma,
             pkA, pkB,
             sem_idx, sem_gA, sem_gB, sem_o):
    wid = lax.axis_index("s") * 2 + lax.axis_index("c")
    lanes = lax.iota(jnp.int32, 16)

    def csplat(v):
        return jnp.full((16,), v, jnp.int32)

    pltpu.sync_copy(depth_hbm, dep_tmp)
    pltpu.sync_copy(ntype_hbm, nt_tmp)
    pltpu.sync_copy(sibling_hbm, sib_tab)
    pltpu.sync_copy(gam_hbm, gam_v)
    pltpu.sync_copy(bet_hbm, bet_v)
    ntrows = [[plsc.load_gather(nt_tmp, [csplat(t), lanes + c * 16])
               for c in range(4)] for t in range(4)]

    def build_depnt(dep, _):
        for c in range(4):
            dchunk = plsc.load_gather(dep_tmp, [csplat(dep), lanes + c * 16])
            for t in range(4):
                plsc.store_scatter(depnt_tab,
                                   [csplat(dep * 4 + t), lanes + c * 16],
                                   dchunk + ntrows[t][c])
        return 0

    lax.fori_loop(0, 64, build_depnt, 0, unroll=False)

    def fire_idx(k, tid_v, ntv, pk):
        base = wid * TW + k * C
        pltpu.async_copy(tid_hbm.at[pl.ds(base, C)], tid_v, sem_idx)
        pltpu.async_copy(nt_hbm.at[pl.ds(base, C)], ntv, sem_idx)
        pltpu.async_copy(pk_hbm.at[pl.ds(base, C)], pk, sem_idx)

    def wait_idx(tid_v, ntv, pk):
        for r in (tid_v, ntv, pk):
            pltpu.make_async_copy(tid_hbm.at[pl.ds(0, C)], r, sem_idx).wait()

    def prep(tid_v, ntv, cidx):
        def prep_g(g, _):
            t = tid_v[pl.ds(g * 16, 16)]
            n = ntv[pl.ds(g * 16, 16)]
            is_key = (n == 0) | (n == 2)
            cidx[pl.ds(g * 16, 16)] = jnp.where(is_key, t, t + KEY_V)
            return 0

        lax.fori_loop(0, NG, prep_g, 0, unroll=True)

    GSPLIT = 4
    GQ = C // GSPLIT

    def fire_gather(cidx, rows, sem):
        for qq in range(GSPLIT):
            pltpu.async_copy(cat_hbm.at[cidx.at[pl.ds(qq * GQ, GQ)]],
                             rows.at[pl.ds(qq * GQ, GQ)], sem)

    def wait_gather(cidx, rows, sem):
        for qq in range(GSPLIT):
            pltpu.make_async_copy(cat_hbm.at[cidx.at[pl.ds(qq * GQ, GQ)]],
                                  rows.at[pl.ds(qq * GQ, GQ)], sem).wait()

    def wait_out():
        pltpu.make_async_copy(out_dma, out_hbm.at[pl.ds(0, C)], sem_o).wait()

    def compute(k, rows, pk):
        g4 = [gam_v[pl.ds(c * 16, 16)] for c in range(4)]
        b4 = [bet_v[pl.ds(c * 16, 16)] for c in range(4)]

        def grp(g, _):
            w16 = pk[pl.ds(g * 16, 16)]
            dn16 = ((w16 >> 10) & 63) * 4 + (w16 & 3)
            sb16 = (w16 >> 2) & 255
            for u in range(16):
                tsp = csplat(g * 16 + u)
                dnsp = csplat(dn16[u])
                sbsp = csplat(sb16[u])
                xs = []
                for c in range(4):
                    col = lanes + c * 16
                    v = plsc.load_gather(rows, [tsp, col])
                    v = v + plsc.load_gather(depnt_tab, [dnsp, col])
                    v = v + plsc.load_gather(sib_tab, [sbsp, col])
                    xs.append(v)
                s = (xs[0] + xs[1]) + (xs[2] + xs[3])
                q = ((xs[0] * xs[0] + xs[1] * xs[1])
                     + (xs[2] * xs[2] + xs[3] * xs[3]))
                mu = jnp.full((16,), jnp.sum(s), jnp.float32) * (1.0 / D)
                msq = jnp.full((16,), jnp.sum(q), jnp.float32) * (1.0 / D)
                var = msq - mu * mu + EPS
                y = lax.bitcast_convert_type(
                    jnp.int32(0x5F3759DF)
                    - lax.shift_right_arithmetic(
                        lax.bitcast_convert_type(var, jnp.int32), 1),
                    jnp.float32)
                y = y * (1.5 - 0.5 * var * y * y)
                y = y * (1.5 - 0.5 * var * y * y)
                y = y * (1.5 - 0.5 * var * y * y)
                for c in range(4):
                    rg = y * g4[c]
                    bc = b4[c] - mu * rg
                    plsc.store_scatter(out_dma, [tsp, lanes + c * 16],
                                       xs[c] * rg + bc)
            return 0

        lax.fori_loop(0, NG, grp, 0, unroll=False)

        @pl.when(k > 0)
        def _():
            wait_out()

        base = wid * TW + k * C
        pltpu.async_copy(out_dma, out_hbm.at[pl.ds(base, C)], sem_o)

    fire_idx(0, tidA, ntA, pkA)
    wait_idx(tidA, ntA, pkA)
    prep(tidA, ntA, cidxA)
    fire_gather(cidxA, rowsA, sem_gA)
    fire_idx(1, tidB, ntB, pkB)

    def pair(kk, _):
        k0 = 2 * kk
        wait_idx(tidB, ntB, pkB)
        prep(tidB, ntB, cidxB)
        fire_gather(cidxB, rowsB, sem_gB)
        wait_gather(cidxA, rowsA, sem_gA)
        compute(k0, rowsA, pkA)

        @pl.when(kk < NP - 1)
        def _():
            fire_idx(k0 + 2, tidA, ntA, pkA)
            wait_idx(tidA, ntA, pkA)
            prep(tidA, ntA, cidxA)
            fire_gather(cidxA, rowsA, sem_gA)

        wait_gather(cidxB, rowsB, sem_gB)
        compute(k0 + 1, rowsB, pkB)

        @pl.when(kk < NP - 1)
        def _():
            fire_idx(k0 + 3, tidB, ntB, pkB)

        return 0

    lax.fori_loop(0, NP, pair, 0, unroll=False)
    wait_out()


_sc_embed = functools.partial(
    pl.kernel,
    out_type=jax.ShapeDtypeStruct((N, D), jnp.float32),
    mesh=plsc.VectorSubcoreMesh(core_axis_name="c", subcore_axis_name="s"),
    compiler_params=pltpu.CompilerParams(
        needs_layout_passes=False, use_tc_tiling_on_sc=False),
    scratch_types=[
        pltpu.VMEM((256, D), jnp.float32),
        pltpu.VMEM((256, D), jnp.float32),
        pltpu.VMEM((64, D), jnp.float32),
        pltpu.VMEM((4, D), jnp.float32),
        pltpu.VMEM((D,), jnp.float32),
        pltpu.VMEM((D,), jnp.float32),
        pltpu.VMEM((C,), jnp.int32),
        pltpu.VMEM((C,), jnp.int32),
        pltpu.VMEM((C,), jnp.int32),
        pltpu.VMEM((C,), jnp.int32),
        pltpu.VMEM((C,), jnp.int32),
        pltpu.VMEM((C,), jnp.int32),
        pltpu.VMEM((C, D), jnp.float32),
        pltpu.VMEM((C, D), jnp.float32),
        pltpu.VMEM((C, D), jnp.float32),
        pltpu.VMEM((C,), jnp.int32),
        pltpu.VMEM((C,), jnp.int32),
        pltpu.SemaphoreType.DMA,
        pltpu.SemaphoreType.DMA,
        pltpu.SemaphoreType.DMA,
        pltpu.SemaphoreType.DMA,
    ],
)(_sc_body)


def kernel(token_ids, node_types, depths, sibling_indices, key_table,
           value_table, depth_table, sibling_table, node_type_table,
           gamma, beta):
    tid = token_ids.reshape(N).astype(jnp.int32)
    nt = node_types.reshape(N).astype(jnp.int32)
    dep = depths.reshape(N).astype(jnp.int32)
    sib = sibling_indices.reshape(N).astype(jnp.int32)
    packed = ((tid & 1) << 16) | (dep << 10) | (sib << 2) | nt
    cat = jnp.concatenate([key_table.astype(jnp.float32),
                           value_table.astype(jnp.float32)], axis=0)
    out = _sc_embed(tid, nt, packed, cat,
                    depth_table.astype(jnp.float32),
                    sibling_table.astype(jnp.float32),
                    node_type_table.astype(jnp.float32),
                    gamma.astype(jnp.float32),
                    beta.astype(jnp.float32))
    return out.reshape(B, L, D)
